# baseline (speedup 1.0000x reference)
import jax
import jax.numpy as jnp
from jax import lax
from jax.experimental import pallas as pl
from jax.experimental.pallas import tpu as pltpu

_N_USERS = 100000
_N_ITEMS = 1000000
_EMB = 64
_BR = 1000
_NS = 22
_RPS = 50
_UBLK = _N_USERS // _BR


def _src_of(s, r):
    g = _RPS * s + r
    if g < _UBLK:
        return 0, g * _BR
    return 1, (g - _UBLK) * _BR


def _body(u_ref, i_ref, o_ref, bufs, rsem, wsem):
    srcs = (u_ref, i_ref)

    def read(s, b, r):
        which, row = _src_of(s, r)
        pltpu.make_async_copy(
            srcs[which].at[pl.ds(row, _BR)], bufs.at[s, b], rsem.at[s, b]
        ).start()

    def wait_read(s, b, r):
        which, row = _src_of(s, r)
        pltpu.make_async_copy(
            srcs[which].at[pl.ds(row, _BR)], bufs.at[s, b], rsem.at[s, b]
        ).wait()

    def write(s, b, r):
        g = _RPS * s + r
        pltpu.make_async_copy(
            bufs.at[s, b], o_ref.at[pl.ds(g * _BR, _BR)], wsem.at[s, b]
        ).start()

    def wait_write(s, b, r):
        g = _RPS * s + r
        pltpu.make_async_copy(
            bufs.at[s, b], o_ref.at[pl.ds(g * _BR, _BR)], wsem.at[s, b]
        ).wait()

    for s in range(_NS):
        read(s, 0, 0)
    for s in range(_NS):
        read(s, 1, 1)

    for k in range(_RPS // 2):
        for b in (0, 1):
            r = 2 * k + b
            for s in range(_NS):
                wait_read(s, b, r)
                write(s, b, r)
            if r + 2 < _RPS:
                for s in range(_NS):
                    wait_write(s, b, r)
                    read(s, b, r + 2)
    for s in range(_NS):
        wait_write(s, 0, _RPS - 2)
        wait_write(s, 1, _RPS - 1)


def kernel(user_emb, item_emb):
    return pl.pallas_call(
        _body,
        out_shape=jax.ShapeDtypeStruct((_N_USERS + _N_ITEMS, _EMB), jnp.float32),
        in_specs=[
            pl.BlockSpec(memory_space=pl.ANY),
            pl.BlockSpec(memory_space=pl.ANY),
        ],
        out_specs=pl.BlockSpec(memory_space=pl.ANY),
        scratch_shapes=[
            pltpu.VMEM((_NS, 2, _BR, _EMB), jnp.float32),
            pltpu.SemaphoreType.DMA((_NS, 2)),
            pltpu.SemaphoreType.DMA((_NS, 2)),
        ],
    )(user_emb, item_emb)


# speedup vs baseline: 5.3378x; 5.3378x over previous
import jax
import jax.numpy as jnp
from jax.experimental import pallas as pl
from jax.experimental.pallas import tpu as pltpu

_N_USERS = 100000
_N_ITEMS = 1000000
_EMB = 64
_SHIFT = _N_USERS % 128
_W = 9088
_UA = 99968 // _W
_NTOT = _N_USERS + _N_ITEMS
_GRID = _UA + -(-(_NTOT - _UA * _W) // _W)


def _body(u_ref, i_ref, o_ref, tail_ref):
    i = pl.program_id(0)

    @pl.when(i < _UA)
    def _():
        o_ref[...] = u_ref[...]

    @pl.when(i >= _UA)
    def _():
        blk = i_ref[...]
        tail = jnp.where(i == _UA, u_ref[:, : _SHIFT], tail_ref[...])
        o_ref[...] = jnp.concatenate([tail, blk[:, : _W - _SHIFT]], axis=1)
        tail_ref[...] = blk[:, _W - _SHIFT :]


def kernel(user_emb, item_emb):
    out_t = pl.pallas_call(
        _body,
        grid=(_GRID,),
        out_shape=jax.ShapeDtypeStruct((_EMB, _NTOT), jnp.float32),
        in_specs=[
            pl.BlockSpec((_EMB, _W), lambda i: (0, jnp.minimum(i, _UA))),
            pl.BlockSpec((_EMB, _W), lambda i: (0, jnp.maximum(i - _UA, 0))),
        ],
        out_specs=pl.BlockSpec((_EMB, _W), lambda i: (0, i)),
        scratch_shapes=[pltpu.VMEM((_EMB, _SHIFT), jnp.float32)],
    )(user_emb.T, item_emb.T)
    return out_t.T
